# Initial kernel scaffold; baseline (speedup 1.0000x reference)
import functools

import jax
import jax.numpy as jnp
from jax import lax
from jax.experimental import pallas as pl
from jax.experimental.pallas import tpu as pltpu
from jax.experimental.pallas import tpu_sc as plsc

N = 10000
D = 128
NPAD = 10240
E = 320000
ECHUNK = 128
EROWS = 2528
EPAD = EROWS * ECHUNK
NSC = 2
NTILE = 16
ROWS_B = EROWS // NTILE
ROWS_C = EROWS // (NSC * NTILE)
ZCH = NPAD // NTILE // ECHUNK
EPS = 1e-5

TCGRID = 8
TCROWS = NPAD // TCGRID


def _row_spec(cols):
    return pl.BlockSpec((TCROWS, cols), lambda i: (i, 0))


def _full_spec(rows, cols):
    return pl.BlockSpec((rows, cols), lambda i: (0, 0))


def _f32(shape):
    return jax.ShapeDtypeStruct(shape, jnp.float32)


@functools.lru_cache(maxsize=None)
def _build():
    mesh = plsc.VectorSubcoreMesh(core_axis_name="c", subcore_axis_name="s")

    @functools.partial(
        pl.kernel,
        out_type=_f32((NSC * NTILE, NPAD)),
        mesh=mesh,
        scratch_types=[
            pltpu.VMEM((NPAD,), jnp.float32),
            pltpu.VMEM((ROWS_C, ECHUNK), jnp.int32),
        ],
    )
    def sc_degree(dst_hbm, out_hbm, cnt_v, idx_v):
        c = lax.axis_index("c")
        s = lax.axis_index("s")
        wid = c * NTILE + s

        def zero_body(i, _):
            cnt_v[pl.ds(i * 16, 16)] = jnp.zeros((16,), jnp.float32)
            return 0

        lax.fori_loop(0, NPAD // 16, zero_body, 0)

        pltpu.sync_copy(dst_hbm.at[pl.ds(wid * ROWS_C, ROWS_C)], idx_v)

        ones = jnp.ones((16,), jnp.float32)

        def cnt_body(k, _):
            for j in range(ECHUNK // 16):
                iv = idx_v[k, pl.ds(j * 16, 16)]
                plsc.addupdate_scatter(cnt_v, [iv], ones)
            return 0

        lax.fori_loop(0, ROWS_C, cnt_body, 0)

        pltpu.sync_copy(cnt_v, out_hbm.at[wid])

    def make_edge_pass(split_features):
        rows_per_tile = ROWS_B if split_features else ROWS_C

        @functools.partial(
            pl.kernel,
            out_type=_f32((NSC, NPAD, D)),
            mesh=mesh,
            scratch_types=[
                pltpu.VMEM((rows_per_tile, ECHUNK), jnp.int32),
                pltpu.VMEM((rows_per_tile, ECHUNK), jnp.int32),
                pltpu.VMEM((ECHUNK, D), jnp.float32),
                pltpu.VMEM_SHARED((NPAD, D), jnp.float32),
                pltpu.SemaphoreType.DMA,
            ],
        )
        def edge_pass(table_hbm, src_hbm, dst_hbm, out_hbm,
                      sidx_v, didx_v, rows_v, acc_sh, sem):
            c = lax.axis_index("c")
            s = lax.axis_index("s")
            if split_features:
                row0 = s * rows_per_tile
            else:
                row0 = (c * NTILE + s) * rows_per_tile

            pltpu.sync_copy(src_hbm.at[pl.ds(row0, rows_per_tile)], sidx_v)
            pltpu.sync_copy(dst_hbm.at[pl.ds(row0, rows_per_tile)], didx_v)

            if split_features:
                off = c * NPAD

                def off_body(k, _):
                    for j in range(ECHUNK // 16):
                        sl = pl.ds(j * 16, 16)
                        sidx_v[k, sl] = sidx_v[k, sl] + off
                    return 0

                lax.fori_loop(0, rows_per_tile, off_body, 0)

            def zero_body(i, _):
                for j in range(D // 16):
                    rows_v[i, pl.ds(j * 16, 16)] = jnp.zeros((16,), jnp.float32)
                return 0

            lax.fori_loop(0, ECHUNK, zero_body, 0)
            for t in range(ZCH):
                rbase = (s * ZCH + t) * ECHUNK
                pltpu.sync_copy(rows_v, acc_sh.at[pl.ds(rbase, ECHUNK)])
            plsc.subcore_barrier()

            def chunk_body(k, _):
                pltpu.async_copy(table_hbm.at[sidx_v.at[k]], rows_v, sem).wait()
                pltpu.sync_copy(rows_v, acc_sh.at[didx_v.at[k]], add=True)
                return 0

            lax.fori_loop(0, rows_per_tile, chunk_body, 0)
            plsc.subcore_barrier()

            for t in range(ZCH):
                rbase = (s * ZCH + t) * ECHUNK
                pltpu.sync_copy(acc_sh.at[pl.ds(rbase, ECHUNK)], rows_v)
                pltpu.sync_copy(rows_v, out_hbm.at[c, pl.ds(rbase, ECHUNK)])

        return edge_pass

    edge_pass_b = make_edge_pass(True)
    edge_pass_c = make_edge_pass(False)

    def tc1_body(degp_ref, x_ref, h_ref, y1a_ref, y1b_ref, dinv_ref):
        deg = jnp.sum(degp_ref[...], axis=1, keepdims=True) + 1.0
        dinv = lax.rsqrt(deg)
        y1a_ref[...] = x_ref[...] * dinv
        y1b_ref[...] = h_ref[...] * dinv
        dinv_ref[...] = jnp.broadcast_to(dinv, (TCROWS, D))

    tc1 = pl.pallas_call(
        tc1_body,
        grid=(TCGRID,),
        in_specs=[_row_spec(NSC * NTILE), _row_spec(D), _row_spec(D)],
        out_specs=[_row_spec(D)] * 3,
        out_shape=[_f32((NPAD, D))] * 3,
    )

    def tc2_body(aggA_ref, aggB_ref, y1a_ref, y1b_ref, dinv_ref, h_ref,
                 Wz_ref, Wr_ref, Wc_ref, bz_ref, br_ref,
                 z_ref, y2_ref, t1_ref):
        dinv = dinv_ref[...]
        aggA = (aggA_ref[...] + y1a_ref[...]) * dinv
        aggB = (aggB_ref[...] + y1b_ref[...]) * dinv
        Wz = Wz_ref[...]
        Wr = Wr_ref[...]
        dot = functools.partial(jnp.dot, preferred_element_type=jnp.float32)
        z = jax.nn.sigmoid(dot(aggA, Wz[:D]) + dot(aggB, Wz[D:]) + bz_ref[...])
        r = jax.nn.sigmoid(dot(aggA, Wr[:D]) + dot(aggB, Wr[D:]) + br_ref[...])
        z_ref[...] = z
        y2_ref[...] = dinv * (r * h_ref[...])
        t1_ref[...] = dot(aggA, Wc_ref[:D])

    tc2 = pl.pallas_call(
        tc2_body,
        grid=(TCGRID,),
        in_specs=[_row_spec(D)] * 6
        + [_full_spec(2 * D, D)] * 3
        + [_full_spec(1, D)] * 2,
        out_specs=[_row_spec(D)] * 3,
        out_shape=[_f32((NPAD, D))] * 3,
    )

    def tc3_body(p0_ref, p1_ref, y2_ref, dinv_ref, t1_ref, z_ref, h_ref,
                 Wc_ref, bc_ref, lnw_ref, lnb_ref, out_ref):
        dot = functools.partial(jnp.dot, preferred_element_type=jnp.float32)
        agg2 = (p0_ref[...] + p1_ref[...] + y2_ref[...]) * dinv_ref[...]
        cand = jnp.tanh(t1_ref[...] + dot(agg2, Wc_ref[D:]) + bc_ref[...])
        z = z_ref[...]
        hn = (1.0 - z) * h_ref[...] + z * cand
        mu = jnp.mean(hn, axis=1, keepdims=True)
        var = jnp.mean((hn - mu) ** 2, axis=1, keepdims=True)
        out_ref[...] = (hn - mu) * lax.rsqrt(var + EPS) * lnw_ref[...] + lnb_ref[...]

    tc3 = pl.pallas_call(
        tc3_body,
        grid=(TCGRID,),
        in_specs=[_row_spec(D)] * 7
        + [_full_spec(2 * D, D)]
        + [_full_spec(1, D)] * 3,
        out_specs=_row_spec(D),
        out_shape=_f32((NPAD, D)),
    )

    return sc_degree, edge_pass_b, edge_pass_c, tc1, tc2, tc3


def kernel(x, edge_index, h_prev, Wz, bz, Wr, br, Wc, bc, ln_w, ln_b):
    sc_degree, edge_pass_b, edge_pass_c, tc1, tc2, tc3 = _build()
    bsz, n, _ = x.shape
    x2 = x.reshape(n, D)
    h2 = h_prev.reshape(n, D)
    xp = jnp.pad(x2, ((0, NPAD - n), (0, 0)))
    hp = jnp.pad(h2, ((0, NPAD - n), (0, 0)))
    src = jnp.pad(edge_index[0], (0, EPAD - E), constant_values=n)
    dst = jnp.pad(edge_index[1], (0, EPAD - E), constant_values=n)
    src = src.reshape(EROWS, ECHUNK)
    dst = dst.reshape(EROWS, ECHUNK)

    degp = sc_degree(dst)
    y1a, y1b, dinvc = tc1(degp.T, xp, hp)
    y1s = jnp.concatenate([y1a, y1b], axis=0)
    aggB = edge_pass_b(y1s, src, dst)
    z, y2, t1 = tc2(aggB[0], aggB[1], y1a, y1b, dinvc, hp, Wz, Wr, Wc,
                    bz.reshape(1, D), br.reshape(1, D))
    aggC = edge_pass_c(y2, src, dst)
    hout = tc3(aggC[0], aggC[1], y2, dinvc, t1, z, hp, Wc,
               bc.reshape(1, D), ln_w.reshape(1, D), ln_b.reshape(1, D))
    return hout[:n].reshape(bsz, n, D)


# speedup vs baseline: 7.5248x; 7.5248x over previous
import functools

import jax
import jax.numpy as jnp
from jax import lax
from jax.experimental import pallas as pl
from jax.experimental.pallas import tpu as pltpu
from jax.experimental.pallas import tpu_sc as plsc

N = 10000
D = 128
NPAD = 10240
E = 320000
ECHUNK = 128
EROWS = 2560
EPAD = EROWS * ECHUNK
NSC = 2
NTILE = 16
ROWS_B = EROWS // NTILE
ROWS_C = EROWS // (NSC * NTILE)
ZCH = NPAD // NTILE // ECHUNK
EPS = 1e-5

TCGRID = 8
TCROWS = NPAD // TCGRID


def _row_spec(cols):
    return pl.BlockSpec((TCROWS, cols), lambda i: (i, 0))


def _full_spec(rows, cols):
    return pl.BlockSpec((rows, cols), lambda i: (0, 0))


def _f32(shape):
    return jax.ShapeDtypeStruct(shape, jnp.float32)


@functools.lru_cache(maxsize=None)
def _build():
    mesh = plsc.VectorSubcoreMesh(core_axis_name="c", subcore_axis_name="s")
    sc_params = pltpu.CompilerParams(needs_layout_passes=False)

    @functools.partial(
        pl.kernel,
        out_type=_f32((NSC * NTILE, 1, NPAD)),
        mesh=mesh,
        compiler_params=sc_params,
        scratch_types=[
            pltpu.VMEM((NPAD,), jnp.float32),
            pltpu.VMEM((ROWS_C, ECHUNK), jnp.int32),
        ],
    )
    def sc_degree(dst_hbm, out_hbm, cnt_v, idx_v):
        c = lax.axis_index("c")
        s = lax.axis_index("s")
        wid = c * NTILE + s

        def zero_body(i, _):
            cnt_v[pl.ds(i * 16, 16)] = jnp.zeros((16,), jnp.float32)
            return 0

        lax.fori_loop(0, NPAD // 16, zero_body, 0)

        pltpu.sync_copy(dst_hbm.at[pl.ds(wid * ROWS_C, ROWS_C)], idx_v)

        ones = jnp.ones((16,), jnp.float32)

        def cnt_body(k, _):
            for j in range(ECHUNK // 16):
                iv = idx_v[k, pl.ds(j * 16, 16)]
                plsc.addupdate_scatter(cnt_v, [iv], ones)
            return 0

        lax.fori_loop(0, ROWS_C, cnt_body, 0)

        pltpu.sync_copy(cnt_v, out_hbm.at[wid, 0])

    def make_edge_pass(split_features):
        rows_per_tile = ROWS_B if split_features else ROWS_C
        IBLK = 8
        nblocks = rows_per_tile // IBLK

        @functools.partial(
            pl.kernel,
            out_type=_f32((NSC, NPAD, D)),
            mesh=mesh,
            compiler_params=sc_params,
            scratch_types=[
                pltpu.VMEM((IBLK, ECHUNK), jnp.int32),
                pltpu.VMEM((IBLK, ECHUNK), jnp.int32),
                pltpu.VMEM((ECHUNK, D), jnp.float32),
                pltpu.VMEM_SHARED((NPAD, D), jnp.float32),
                pltpu.SemaphoreType.DMA,
            ],
        )
        def edge_pass(table_hbm, src_hbm, dst_hbm, out_hbm,
                      sidx_v, didx_v, rows_v, acc_sh, sem):
            c = lax.axis_index("c")
            s = lax.axis_index("s")
            if split_features:
                row0 = s * rows_per_tile
            else:
                row0 = (c * NTILE + s) * rows_per_tile
            off = c * NPAD

            def zero_body(i, _):
                for j in range(D // 16):
                    rows_v[i, pl.ds(j * 16, 16)] = jnp.zeros((16,), jnp.float32)
                return 0

            lax.fori_loop(0, ECHUNK, zero_body, 0)
            for t in range(ZCH):
                rbase = (s * ZCH + t) * ECHUNK
                pltpu.sync_copy(rows_v, acc_sh.at[pl.ds(rbase, ECHUNK)])
            plsc.subcore_barrier()

            def block_body(b, _):
                rb = row0 + b * IBLK
                pltpu.sync_copy(src_hbm.at[pl.ds(rb, IBLK)], sidx_v)
                pltpu.sync_copy(dst_hbm.at[pl.ds(rb, IBLK)], didx_v)
                if split_features:
                    for k in range(IBLK):
                        for j in range(ECHUNK // 16):
                            sl = pl.ds(j * 16, 16)
                            sidx_v[k, sl] = sidx_v[k, sl] + off
                for k in range(IBLK):
                    pltpu.async_copy(
                        table_hbm.at[sidx_v.at[k]], rows_v, sem).wait()
                    pltpu.sync_copy(rows_v, acc_sh.at[didx_v.at[k]], add=True)
                return 0

            lax.fori_loop(0, nblocks, block_body, 0)
            plsc.subcore_barrier()

            for t in range(ZCH):
                rbase = (s * ZCH + t) * ECHUNK
                pltpu.sync_copy(acc_sh.at[pl.ds(rbase, ECHUNK)], rows_v)
                pltpu.sync_copy(rows_v, out_hbm.at[c, pl.ds(rbase, ECHUNK)])

        return edge_pass

    edge_pass_b = make_edge_pass(True)
    edge_pass_c = make_edge_pass(False)

    def tc1_body(degp_ref, x_ref, h_ref, y1a_ref, y1b_ref, dinv_ref):
        deg = jnp.sum(degp_ref[...], axis=1, keepdims=True) + 1.0
        dinv = lax.rsqrt(deg)
        y1a_ref[...] = x_ref[...] * dinv
        y1b_ref[...] = h_ref[...] * dinv
        dinv_ref[...] = jnp.broadcast_to(dinv, (TCROWS, D))

    tc1 = pl.pallas_call(
        tc1_body,
        grid=(TCGRID,),
        in_specs=[_row_spec(NSC * NTILE), _row_spec(D), _row_spec(D)],
        out_specs=[_row_spec(D)] * 3,
        out_shape=[_f32((NPAD, D))] * 3,
    )

    def tc2_body(aggA_ref, aggB_ref, y1a_ref, y1b_ref, dinv_ref, h_ref,
                 Wz_ref, Wr_ref, Wc_ref, bz_ref, br_ref,
                 z_ref, y2_ref, t1_ref):
        dinv = dinv_ref[...]
        aggA = (aggA_ref[...] + y1a_ref[...]) * dinv
        aggB = (aggB_ref[...] + y1b_ref[...]) * dinv
        Wz = Wz_ref[...]
        Wr = Wr_ref[...]
        dot = functools.partial(jnp.dot, preferred_element_type=jnp.float32)
        z = jax.nn.sigmoid(dot(aggA, Wz[:D]) + dot(aggB, Wz[D:]) + bz_ref[...])
        r = jax.nn.sigmoid(dot(aggA, Wr[:D]) + dot(aggB, Wr[D:]) + br_ref[...])
        z_ref[...] = z
        y2_ref[...] = dinv * (r * h_ref[...])
        t1_ref[...] = dot(aggA, Wc_ref[:D])

    tc2 = pl.pallas_call(
        tc2_body,
        grid=(TCGRID,),
        in_specs=[_row_spec(D)] * 6
        + [_full_spec(2 * D, D)] * 3
        + [_full_spec(1, D)] * 2,
        out_specs=[_row_spec(D)] * 3,
        out_shape=[_f32((NPAD, D))] * 3,
    )

    def tc3_body(p0_ref, p1_ref, y2_ref, dinv_ref, t1_ref, z_ref, h_ref,
                 Wc_ref, bc_ref, lnw_ref, lnb_ref, out_ref):
        dot = functools.partial(jnp.dot, preferred_element_type=jnp.float32)
        agg2 = (p0_ref[...] + p1_ref[...] + y2_ref[...]) * dinv_ref[...]
        cand = jnp.tanh(t1_ref[...] + dot(agg2, Wc_ref[D:]) + bc_ref[...])
        z = z_ref[...]
        hn = (1.0 - z) * h_ref[...] + z * cand
        mu = jnp.mean(hn, axis=1, keepdims=True)
        var = jnp.mean((hn - mu) ** 2, axis=1, keepdims=True)
        out_ref[...] = (hn - mu) * lax.rsqrt(var + EPS) * lnw_ref[...] + lnb_ref[...]

    tc3 = pl.pallas_call(
        tc3_body,
        grid=(TCGRID,),
        in_specs=[_row_spec(D)] * 7
        + [_full_spec(2 * D, D)]
        + [_full_spec(1, D)] * 3,
        out_specs=_row_spec(D),
        out_shape=_f32((NPAD, D)),
    )

    return sc_degree, edge_pass_b, edge_pass_c, tc1, tc2, tc3


def kernel(x, edge_index, h_prev, Wz, bz, Wr, br, Wc, bc, ln_w, ln_b):
    sc_degree, edge_pass_b, edge_pass_c, tc1, tc2, tc3 = _build()
    bsz, n, _ = x.shape
    x2 = x.reshape(n, D)
    h2 = h_prev.reshape(n, D)
    xp = jnp.pad(x2, ((0, NPAD - n), (0, 0)))
    hp = jnp.pad(h2, ((0, NPAD - n), (0, 0)))
    src = jnp.pad(edge_index[0], (0, EPAD - E), constant_values=n)
    dst = jnp.pad(edge_index[1], (0, EPAD - E), constant_values=n)
    src = src.reshape(EROWS, ECHUNK)
    dst = dst.reshape(EROWS, ECHUNK)

    degp = sc_degree(dst)
    y1a, y1b, dinvc = tc1(degp.reshape(NSC * NTILE, NPAD).T, xp, hp)
    y1s = jnp.concatenate([y1a, y1b], axis=0)
    aggB = edge_pass_b(y1s, src, dst)
    z, y2, t1 = tc2(aggB[0], aggB[1], y1a, y1b, dinvc, hp, Wz, Wr, Wc,
                    bz.reshape(1, D), br.reshape(1, D))
    aggC = edge_pass_c(y2, src, dst)
    hout = tc3(aggC[0], aggC[1], y2, dinvc, t1, z, hp, Wc,
               bc.reshape(1, D), ln_w.reshape(1, D), ln_b.reshape(1, D))
    return hout[:n].reshape(bsz, n, D)


# speedup vs baseline: 8.6117x; 1.1444x over previous
import functools

import jax
import jax.numpy as jnp
from jax import lax
from jax.experimental import pallas as pl
from jax.experimental.pallas import tpu as pltpu
from jax.experimental.pallas import tpu_sc as plsc

N = 10000
D = 128
NPAD = 10240
E = 320000
ECHUNK = 128
EROWS = 2560
EPAD = EROWS * ECHUNK
NSC = 2
NTILE = 16
ROWS_B = EROWS // NTILE
ROWS_C = EROWS // (NSC * NTILE)
ZCH = NPAD // NTILE // ECHUNK
EPS = 1e-5

TCGRID = 8
TCROWS = NPAD // TCGRID


def _row_spec(cols):
    return pl.BlockSpec((TCROWS, cols), lambda i: (i, 0))


def _full_spec(rows, cols):
    return pl.BlockSpec((rows, cols), lambda i: (0, 0))


def _f32(shape):
    return jax.ShapeDtypeStruct(shape, jnp.float32)


@functools.lru_cache(maxsize=None)
def _build():
    mesh = plsc.VectorSubcoreMesh(core_axis_name="c", subcore_axis_name="s")
    sc_params = pltpu.CompilerParams(needs_layout_passes=False)

    @functools.partial(
        pl.kernel,
        out_type=_f32((NSC * NTILE, 1, NPAD)),
        mesh=mesh,
        compiler_params=sc_params,
        scratch_types=[
            pltpu.VMEM((NPAD,), jnp.float32),
            pltpu.VMEM((ROWS_C, ECHUNK), jnp.int32),
        ],
    )
    def sc_degree(dst_hbm, out_hbm, cnt_v, idx_v):
        c = lax.axis_index("c")
        s = lax.axis_index("s")
        wid = c * NTILE + s

        def zero_body(i, _):
            cnt_v[pl.ds(i * 16, 16)] = jnp.zeros((16,), jnp.float32)
            return 0

        lax.fori_loop(0, NPAD // 16, zero_body, 0)

        pltpu.sync_copy(dst_hbm.at[pl.ds(wid * ROWS_C, ROWS_C)], idx_v)

        ones = jnp.ones((16,), jnp.float32)

        def cnt_body(k, _):
            for j in range(ECHUNK // 16):
                iv = idx_v[k, pl.ds(j * 16, 16)]
                plsc.addupdate_scatter(cnt_v, [iv], ones)
            return 0

        lax.fori_loop(0, ROWS_C, cnt_body, 0)

        pltpu.sync_copy(cnt_v, out_hbm.at[wid, 0])

    def make_edge_pass(split_features):
        rows_per_tile = ROWS_B if split_features else ROWS_C
        IBLK = 16
        nblocks = rows_per_tile // IBLK

        @functools.partial(
            pl.kernel,
            out_type=_f32((NSC, NPAD, D)),
            mesh=mesh,
            compiler_params=sc_params,
            scratch_types=[
                pltpu.VMEM((IBLK, ECHUNK), jnp.int32),
                pltpu.VMEM((IBLK, ECHUNK), jnp.int32),
                pltpu.VMEM((ECHUNK, D), jnp.float32),
                pltpu.VMEM((ECHUNK, D), jnp.float32),
                pltpu.VMEM_SHARED((NPAD, D), jnp.float32),
                pltpu.SemaphoreType.DMA,
                pltpu.SemaphoreType.DMA,
            ],
        )
        def edge_pass(table_hbm, src_hbm, dst_hbm, out_hbm,
                      sidx_v, didx_v, rows0_v, rows1_v, acc_sh, sem0, sem1):
            c = lax.axis_index("c")
            s = lax.axis_index("s")
            if split_features:
                row0 = s * rows_per_tile
                table = table_hbm.at[pl.ds(c * NPAD, NPAD)]
            else:
                row0 = (c * NTILE + s) * rows_per_tile
                table = table_hbm

            bufs = (rows0_v, rows1_v)
            sems = (sem0, sem1)

            def zero_body(i, _):
                for j in range(D // 16):
                    rows0_v[i, pl.ds(j * 16, 16)] = jnp.zeros((16,), jnp.float32)
                return 0

            lax.fori_loop(0, ECHUNK, zero_body, 0)
            for t in range(ZCH):
                rbase = (s * ZCH + t) * ECHUNK
                pltpu.sync_copy(rows0_v, acc_sh.at[pl.ds(rbase, ECHUNK)])
            plsc.subcore_barrier()

            def block_body(b, _):
                rb = row0 + b * IBLK
                pltpu.sync_copy(src_hbm.at[pl.ds(rb, IBLK)], sidx_v)
                pltpu.sync_copy(dst_hbm.at[pl.ds(rb, IBLK)], didx_v)
                desc = pltpu.async_copy(table.at[sidx_v.at[0]], bufs[0], sems[0])
                for k in range(IBLK):
                    p = k % 2
                    if k + 1 < IBLK:
                        nxt = pltpu.async_copy(
                            table.at[sidx_v.at[k + 1]], bufs[1 - p], sems[1 - p])
                    desc.wait()
                    pltpu.sync_copy(bufs[p], acc_sh.at[didx_v.at[k]], add=True)
                    if k + 1 < IBLK:
                        desc = nxt
                return 0

            lax.fori_loop(0, nblocks, block_body, 0)
            plsc.subcore_barrier()

            for t in range(ZCH):
                rbase = (s * ZCH + t) * ECHUNK
                pltpu.sync_copy(acc_sh.at[pl.ds(rbase, ECHUNK)], rows0_v)
                pltpu.sync_copy(rows0_v, out_hbm.at[c, pl.ds(rbase, ECHUNK)])

        return edge_pass

    edge_pass_b = make_edge_pass(True)
    edge_pass_c = make_edge_pass(False)

    def tc1_body(degp_ref, x_ref, h_ref, y1a_ref, y1b_ref, dinv_ref):
        deg = jnp.sum(degp_ref[...], axis=1, keepdims=True) + 1.0
        dinv = lax.rsqrt(deg)
        y1a_ref[...] = x_ref[...] * dinv
        y1b_ref[...] = h_ref[...] * dinv
        dinv_ref[...] = jnp.broadcast_to(dinv, (TCROWS, D))

    tc1 = pl.pallas_call(
        tc1_body,
        grid=(TCGRID,),
        in_specs=[_row_spec(NSC * NTILE), _row_spec(D), _row_spec(D)],
        out_specs=[_row_spec(D)] * 3,
        out_shape=[_f32((NPAD, D))] * 3,
    )

    def tc2_body(aggA_ref, aggB_ref, y1a_ref, y1b_ref, dinv_ref, h_ref,
                 Wz_ref, Wr_ref, Wc_ref, bz_ref, br_ref,
                 z_ref, y2_ref, t1_ref):
        dinv = dinv_ref[...]
        aggA = (aggA_ref[...] + y1a_ref[...]) * dinv
        aggB = (aggB_ref[...] + y1b_ref[...]) * dinv
        Wz = Wz_ref[...]
        Wr = Wr_ref[...]
        dot = functools.partial(jnp.dot, preferred_element_type=jnp.float32)
        z = jax.nn.sigmoid(dot(aggA, Wz[:D]) + dot(aggB, Wz[D:]) + bz_ref[...])
        r = jax.nn.sigmoid(dot(aggA, Wr[:D]) + dot(aggB, Wr[D:]) + br_ref[...])
        z_ref[...] = z
        y2_ref[...] = dinv * (r * h_ref[...])
        t1_ref[...] = dot(aggA, Wc_ref[:D])

    tc2 = pl.pallas_call(
        tc2_body,
        grid=(TCGRID,),
        in_specs=[_row_spec(D)] * 6
        + [_full_spec(2 * D, D)] * 3
        + [_full_spec(1, D)] * 2,
        out_specs=[_row_spec(D)] * 3,
        out_shape=[_f32((NPAD, D))] * 3,
    )

    def tc3_body(p0_ref, p1_ref, y2_ref, dinv_ref, t1_ref, z_ref, h_ref,
                 Wc_ref, bc_ref, lnw_ref, lnb_ref, out_ref):
        dot = functools.partial(jnp.dot, preferred_element_type=jnp.float32)
        agg2 = (p0_ref[...] + p1_ref[...] + y2_ref[...]) * dinv_ref[...]
        cand = jnp.tanh(t1_ref[...] + dot(agg2, Wc_ref[D:]) + bc_ref[...])
        z = z_ref[...]
        hn = (1.0 - z) * h_ref[...] + z * cand
        mu = jnp.mean(hn, axis=1, keepdims=True)
        var = jnp.mean((hn - mu) ** 2, axis=1, keepdims=True)
        out_ref[...] = (hn - mu) * lax.rsqrt(var + EPS) * lnw_ref[...] + lnb_ref[...]

    tc3 = pl.pallas_call(
        tc3_body,
        grid=(TCGRID,),
        in_specs=[_row_spec(D)] * 7
        + [_full_spec(2 * D, D)]
        + [_full_spec(1, D)] * 3,
        out_specs=_row_spec(D),
        out_shape=_f32((NPAD, D)),
    )

    return sc_degree, edge_pass_b, edge_pass_c, tc1, tc2, tc3


def kernel(x, edge_index, h_prev, Wz, bz, Wr, br, Wc, bc, ln_w, ln_b):
    sc_degree, edge_pass_b, edge_pass_c, tc1, tc2, tc3 = _build()
    bsz, n, _ = x.shape
    x2 = x.reshape(n, D)
    h2 = h_prev.reshape(n, D)
    xp = jnp.pad(x2, ((0, NPAD - n), (0, 0)))
    hp = jnp.pad(h2, ((0, NPAD - n), (0, 0)))
    src = jnp.pad(edge_index[0], (0, EPAD - E), constant_values=n)
    dst = jnp.pad(edge_index[1], (0, EPAD - E), constant_values=n)
    src = src.reshape(EROWS, ECHUNK)
    dst = dst.reshape(EROWS, ECHUNK)

    degp = sc_degree(dst)
    y1a, y1b, dinvc = tc1(degp.reshape(NSC * NTILE, NPAD).T, xp, hp)
    y1s = jnp.concatenate([y1a, y1b], axis=0)
    aggB = edge_pass_b(y1s, src, dst)
    z, y2, t1 = tc2(aggB[0], aggB[1], y1a, y1b, dinvc, hp, Wz, Wr, Wc,
                    bz.reshape(1, D), br.reshape(1, D))
    aggC = edge_pass_c(y2, src, dst)
    hout = tc3(aggC[0], aggC[1], y2, dinvc, t1, z, hp, Wc,
               bc.reshape(1, D), ln_w.reshape(1, D), ln_b.reshape(1, D))
    return hout[:n].reshape(bsz, n, D)


# speedup vs baseline: 9.0100x; 1.0463x over previous
import functools

import jax
import jax.numpy as jnp
from jax import lax
from jax.experimental import pallas as pl
from jax.experimental.pallas import tpu as pltpu
from jax.experimental.pallas import tpu_sc as plsc

N = 10000
D = 128
NPAD = 10240
E = 320000
ECHUNK = 64
EROWS = 5120
EPAD = EROWS * ECHUNK
NSC = 2
NTILE = 16
ROWS_B = EROWS // NTILE
ROWS_C = EROWS // (NSC * NTILE)
IBLK = 16
NBUF = 4
NR = NPAD // NTILE
EPS = 1e-5

TCGRID = 8
TCROWS = NPAD // TCGRID


def _row_spec(cols):
    return pl.BlockSpec((TCROWS, cols), lambda i: (i, 0))


def _full_spec(rows, cols):
    return pl.BlockSpec((rows, cols), lambda i: (0, 0))


def _f32(shape):
    return jax.ShapeDtypeStruct(shape, jnp.float32)


@functools.lru_cache(maxsize=None)
def _build():
    mesh = plsc.VectorSubcoreMesh(core_axis_name="c", subcore_axis_name="s")
    sc_params = pltpu.CompilerParams(needs_layout_passes=False)

    @functools.partial(
        pl.kernel,
        out_type=_f32((NSC * NTILE, 1, NPAD)),
        mesh=mesh,
        compiler_params=sc_params,
        scratch_types=[
            pltpu.VMEM((NPAD,), jnp.float32),
            pltpu.VMEM((ROWS_C, ECHUNK), jnp.int32),
        ],
    )
    def sc_degree(dst_hbm, out_hbm, cnt_v, idx_v):
        c = lax.axis_index("c")
        s = lax.axis_index("s")
        wid = c * NTILE + s

        def zero_body(i, _):
            cnt_v[pl.ds(i * 16, 16)] = jnp.zeros((16,), jnp.float32)
            return 0

        lax.fori_loop(0, NPAD // 16, zero_body, 0)

        pltpu.sync_copy(dst_hbm.at[pl.ds(wid * ROWS_C, ROWS_C)], idx_v)

        ones = jnp.ones((16,), jnp.float32)

        def cnt_body(k, _):
            for j in range(ECHUNK // 16):
                iv = idx_v[k, pl.ds(j * 16, 16)]
                plsc.addupdate_scatter(cnt_v, [iv], ones)
            return 0

        lax.fori_loop(0, ROWS_C, cnt_body, 0)

        pltpu.sync_copy(cnt_v, out_hbm.at[wid, 0])

    def make_edge_pass(split_features):
        rows_per_tile = ROWS_B if split_features else ROWS_C
        nblocks = rows_per_tile // IBLK

        @functools.partial(
            pl.kernel,
            out_type=_f32((NSC, NPAD, D)),
            mesh=mesh,
            compiler_params=sc_params,
            scratch_types=[
                pltpu.VMEM((IBLK, ECHUNK), jnp.int32),
                pltpu.VMEM((IBLK, ECHUNK), jnp.int32),
                [pltpu.VMEM((ECHUNK, D), jnp.float32)] * NBUF,
                pltpu.VMEM_SHARED((NPAD, D), jnp.float32),
                [pltpu.SemaphoreType.DMA] * NBUF,
                [pltpu.SemaphoreType.DMA] * NBUF,
            ],
        )
        def edge_pass(table_hbm, src_hbm, dst_hbm, out_hbm,
                      sidx_v, didx_v, bufs, acc_sh, gsems, ssems):
            c = lax.axis_index("c")
            s = lax.axis_index("s")
            if split_features:
                row0 = s * rows_per_tile
                table = table_hbm.at[pl.ds(c * NPAD, NPAD)]
            else:
                row0 = (c * NTILE + s) * rows_per_tile
                table = table_hbm

            def zero_body(i, _):
                for j in range(D // 16):
                    bufs[0][i, pl.ds(j * 16, 16)] = jnp.zeros((16,), jnp.float32)
                return 0

            lax.fori_loop(0, ECHUNK, zero_body, 0)
            for t in range(NR // ECHUNK):
                rbase = s * NR + t * ECHUNK
                pltpu.sync_copy(bufs[0], acc_sh.at[pl.ds(rbase, ECHUNK)])
            plsc.subcore_barrier()

            def block_body(b, _):
                rb = row0 + b * IBLK
                pltpu.sync_copy(src_hbm.at[pl.ds(rb, IBLK)], sidx_v)
                pltpu.sync_copy(dst_hbm.at[pl.ds(rb, IBLK)], didx_v)
                gd = [None] * NBUF
                sd = [None] * NBUF
                for j in range(NBUF - 1):
                    gd[j] = pltpu.async_copy(
                        table.at[sidx_v.at[j]], bufs[j], gsems[j])
                for k in range(IBLK):
                    r = k % NBUF
                    gd[r].wait()
                    sd[r] = pltpu.async_copy(
                        bufs[r], acc_sh.at[didx_v.at[k]], ssems[r], add=True)
                    j = k + NBUF - 1
                    if j < IBLK:
                        rj = j % NBUF
                        if sd[rj] is not None:
                            sd[rj].wait()
                        gd[rj] = pltpu.async_copy(
                            table.at[sidx_v.at[j]], bufs[rj], gsems[rj])
                for dsc in sd:
                    if dsc is not None:
                        dsc.wait()
                return 0

            lax.fori_loop(0, nblocks, block_body, 0)
            plsc.subcore_barrier()

            for t in range(NR // ECHUNK):
                rbase = s * NR + t * ECHUNK
                pltpu.sync_copy(acc_sh.at[pl.ds(rbase, ECHUNK)], bufs[0])
                pltpu.sync_copy(bufs[0], out_hbm.at[c, pl.ds(rbase, ECHUNK)])

        return edge_pass

    edge_pass_b = make_edge_pass(True)
    edge_pass_c = make_edge_pass(False)

    def tc1_body(degp_ref, x_ref, h_ref, y1a_ref, y1b_ref, dinv_ref):
        deg = jnp.sum(degp_ref[...], axis=1, keepdims=True) + 1.0
        dinv = lax.rsqrt(deg)
        y1a_ref[...] = x_ref[...] * dinv
        y1b_ref[...] = h_ref[...] * dinv
        dinv_ref[...] = jnp.broadcast_to(dinv, (TCROWS, D))

    tc1 = pl.pallas_call(
        tc1_body,
        grid=(TCGRID,),
        in_specs=[_row_spec(NSC * NTILE), _row_spec(D), _row_spec(D)],
        out_specs=[_row_spec(D)] * 3,
        out_shape=[_f32((NPAD, D))] * 3,
    )

    def tc2_body(aggA_ref, aggB_ref, y1a_ref, y1b_ref, dinv_ref, h_ref,
                 Wz_ref, Wr_ref, Wc_ref, bz_ref, br_ref,
                 z_ref, y2_ref, t1_ref):
        dinv = dinv_ref[...]
        aggA = (aggA_ref[...] + y1a_ref[...]) * dinv
        aggB = (aggB_ref[...] + y1b_ref[...]) * dinv
        Wz = Wz_ref[...]
        Wr = Wr_ref[...]
        dot = functools.partial(jnp.dot, preferred_element_type=jnp.float32)
        z = jax.nn.sigmoid(dot(aggA, Wz[:D]) + dot(aggB, Wz[D:]) + bz_ref[...])
        r = jax.nn.sigmoid(dot(aggA, Wr[:D]) + dot(aggB, Wr[D:]) + br_ref[...])
        z_ref[...] = z
        y2_ref[...] = dinv * (r * h_ref[...])
        t1_ref[...] = dot(aggA, Wc_ref[:D])

    tc2 = pl.pallas_call(
        tc2_body,
        grid=(TCGRID,),
        in_specs=[_row_spec(D)] * 6
        + [_full_spec(2 * D, D)] * 3
        + [_full_spec(1, D)] * 2,
        out_specs=[_row_spec(D)] * 3,
        out_shape=[_f32((NPAD, D))] * 3,
    )

    def tc3_body(p0_ref, p1_ref, y2_ref, dinv_ref, t1_ref, z_ref, h_ref,
                 Wc_ref, bc_ref, lnw_ref, lnb_ref, out_ref):
        dot = functools.partial(jnp.dot, preferred_element_type=jnp.float32)
        agg2 = (p0_ref[...] + p1_ref[...] + y2_ref[...]) * dinv_ref[...]
        cand = jnp.tanh(t1_ref[...] + dot(agg2, Wc_ref[D:]) + bc_ref[...])
        z = z_ref[...]
        hn = (1.0 - z) * h_ref[...] + z * cand
        mu = jnp.mean(hn, axis=1, keepdims=True)
        var = jnp.mean((hn - mu) ** 2, axis=1, keepdims=True)
        out_ref[...] = (hn - mu) * lax.rsqrt(var + EPS) * lnw_ref[...] + lnb_ref[...]

    tc3 = pl.pallas_call(
        tc3_body,
        grid=(TCGRID,),
        in_specs=[_row_spec(D)] * 7
        + [_full_spec(2 * D, D)]
        + [_full_spec(1, D)] * 3,
        out_specs=_row_spec(D),
        out_shape=_f32((NPAD, D)),
    )

    return sc_degree, edge_pass_b, edge_pass_c, tc1, tc2, tc3


def kernel(x, edge_index, h_prev, Wz, bz, Wr, br, Wc, bc, ln_w, ln_b):
    sc_degree, edge_pass_b, edge_pass_c, tc1, tc2, tc3 = _build()
    bsz, n, _ = x.shape
    x2 = x.reshape(n, D)
    h2 = h_prev.reshape(n, D)
    xp = jnp.pad(x2, ((0, NPAD - n), (0, 0)))
    hp = jnp.pad(h2, ((0, NPAD - n), (0, 0)))
    src = jnp.pad(edge_index[0], (0, EPAD - E), constant_values=n)
    dst = jnp.pad(edge_index[1], (0, EPAD - E), constant_values=n)
    src = src.reshape(EROWS, ECHUNK)
    dst = dst.reshape(EROWS, ECHUNK)

    degp = sc_degree(dst)
    y1a, y1b, dinvc = tc1(degp.reshape(NSC * NTILE, NPAD).T, xp, hp)
    y1s = jnp.concatenate([y1a, y1b], axis=0)
    aggB = edge_pass_b(y1s, src, dst)
    z, y2, t1 = tc2(aggB[0], aggB[1], y1a, y1b, dinvc, hp, Wz, Wr, Wc,
                    bz.reshape(1, D), br.reshape(1, D))
    aggC = edge_pass_c(y2, src, dst)
    hout = tc3(aggC[0], aggC[1], y2, dinvc, t1, z, hp, Wc,
               bc.reshape(1, D), ln_w.reshape(1, D), ln_b.reshape(1, D))
    return hout[:n].reshape(bsz, n, D)


# speedup vs baseline: 9.2517x; 1.0268x over previous
import functools

import jax
import jax.numpy as jnp
from jax import lax
from jax.experimental import pallas as pl
from jax.experimental.pallas import tpu as pltpu
from jax.experimental.pallas import tpu_sc as plsc

N = 10000
D = 128
NPAD = 10240
E = 320000
ECHUNK = 64
EROWS = 5120
EPAD = EROWS * ECHUNK
NSC = 2
NTILE = 16
ROWS_B = EROWS // NTILE
ROWS_C = EROWS // (NSC * NTILE)
IBLK = 32
NBUF = 5
NR = NPAD // NTILE
EPS = 1e-5

TCGRID = 8
TCROWS = NPAD // TCGRID


def _row_spec(cols):
    return pl.BlockSpec((TCROWS, cols), lambda i: (i, 0))


def _full_spec(rows, cols):
    return pl.BlockSpec((rows, cols), lambda i: (0, 0))


def _f32(shape):
    return jax.ShapeDtypeStruct(shape, jnp.float32)


@functools.lru_cache(maxsize=None)
def _build():
    mesh = plsc.VectorSubcoreMesh(core_axis_name="c", subcore_axis_name="s")
    sc_params = pltpu.CompilerParams(needs_layout_passes=False)

    @functools.partial(
        pl.kernel,
        out_type=_f32((NSC * NTILE, 1, NPAD)),
        mesh=mesh,
        compiler_params=sc_params,
        scratch_types=[
            pltpu.VMEM((NPAD,), jnp.float32),
            pltpu.VMEM((ROWS_C, ECHUNK), jnp.int32),
        ],
    )
    def sc_degree(dst_hbm, out_hbm, cnt_v, idx_v):
        c = lax.axis_index("c")
        s = lax.axis_index("s")
        wid = c * NTILE + s

        def zero_body(i, _):
            cnt_v[pl.ds(i * 16, 16)] = jnp.zeros((16,), jnp.float32)
            return 0

        lax.fori_loop(0, NPAD // 16, zero_body, 0)

        pltpu.sync_copy(dst_hbm.at[pl.ds(wid * ROWS_C, ROWS_C)], idx_v)

        ones = jnp.ones((16,), jnp.float32)

        def cnt_body(k, _):
            for j in range(ECHUNK // 16):
                iv = idx_v[k, pl.ds(j * 16, 16)]
                plsc.addupdate_scatter(cnt_v, [iv], ones)
            return 0

        lax.fori_loop(0, ROWS_C, cnt_body, 0)

        pltpu.sync_copy(cnt_v, out_hbm.at[wid, 0])

    def make_edge_pass(split_features):
        rows_per_tile = ROWS_B if split_features else ROWS_C
        nblocks = rows_per_tile // IBLK

        @functools.partial(
            pl.kernel,
            out_type=_f32((NSC, NPAD, D)),
            mesh=mesh,
            compiler_params=sc_params,
            scratch_types=[
                pltpu.VMEM((IBLK, ECHUNK), jnp.int32),
                pltpu.VMEM((IBLK, ECHUNK), jnp.int32),
                [pltpu.VMEM((ECHUNK, D), jnp.float32)] * NBUF,
                pltpu.VMEM_SHARED((NPAD, D), jnp.float32),
                [pltpu.SemaphoreType.DMA] * NBUF,
                [pltpu.SemaphoreType.DMA] * NBUF,
            ],
        )
        def edge_pass(table_hbm, src_hbm, dst_hbm, out_hbm,
                      sidx_v, didx_v, bufs, acc_sh, gsems, ssems):
            c = lax.axis_index("c")
            s = lax.axis_index("s")
            if split_features:
                row0 = s * rows_per_tile
                table = table_hbm.at[pl.ds(c * NPAD, NPAD)]
            else:
                row0 = (c * NTILE + s) * rows_per_tile
                table = table_hbm

            def zero_body(i, _):
                for j in range(D // 16):
                    bufs[0][i, pl.ds(j * 16, 16)] = jnp.zeros((16,), jnp.float32)
                return 0

            lax.fori_loop(0, ECHUNK, zero_body, 0)
            for t in range(NR // ECHUNK):
                rbase = s * NR + t * ECHUNK
                pltpu.sync_copy(bufs[0], acc_sh.at[pl.ds(rbase, ECHUNK)])
            plsc.subcore_barrier()

            def block_body(b, _):
                rb = row0 + b * IBLK
                pltpu.sync_copy(src_hbm.at[pl.ds(rb, IBLK)], sidx_v)
                pltpu.sync_copy(dst_hbm.at[pl.ds(rb, IBLK)], didx_v)
                gd = [None] * NBUF
                sd = [None] * NBUF
                for j in range(NBUF - 1):
                    gd[j] = pltpu.async_copy(
                        table.at[sidx_v.at[j]], bufs[j], gsems[j])
                for k in range(IBLK):
                    r = k % NBUF
                    gd[r].wait()
                    sd[r] = pltpu.async_copy(
                        bufs[r], acc_sh.at[didx_v.at[k]], ssems[r], add=True)
                    j = k + NBUF - 1
                    if j < IBLK:
                        rj = j % NBUF
                        if sd[rj] is not None:
                            sd[rj].wait()
                        gd[rj] = pltpu.async_copy(
                            table.at[sidx_v.at[j]], bufs[rj], gsems[rj])
                for dsc in sd:
                    if dsc is not None:
                        dsc.wait()
                return 0

            lax.fori_loop(0, nblocks, block_body, 0)
            plsc.subcore_barrier()

            for t in range(NR // ECHUNK):
                rbase = s * NR + t * ECHUNK
                pltpu.sync_copy(acc_sh.at[pl.ds(rbase, ECHUNK)], bufs[0])
                pltpu.sync_copy(bufs[0], out_hbm.at[c, pl.ds(rbase, ECHUNK)])

        return edge_pass

    edge_pass_b = make_edge_pass(True)
    edge_pass_c = make_edge_pass(False)

    def tc1_body(degp_ref, x_ref, h_ref, y1a_ref, y1b_ref, dinv_ref):
        deg = jnp.sum(degp_ref[...], axis=1, keepdims=True) + 1.0
        dinv = lax.rsqrt(deg)
        y1a_ref[...] = x_ref[...] * dinv
        y1b_ref[...] = h_ref[...] * dinv
        dinv_ref[...] = jnp.broadcast_to(dinv, (TCROWS, D))

    tc1 = pl.pallas_call(
        tc1_body,
        grid=(TCGRID,),
        in_specs=[_row_spec(NSC * NTILE), _row_spec(D), _row_spec(D)],
        out_specs=[_row_spec(D)] * 3,
        out_shape=[_f32((NPAD, D))] * 3,
    )

    def tc2_body(aggA_ref, aggB_ref, y1a_ref, y1b_ref, dinv_ref, h_ref,
                 Wz_ref, Wr_ref, Wc_ref, bz_ref, br_ref,
                 z_ref, y2_ref, t1_ref):
        dinv = dinv_ref[...]
        aggA = (aggA_ref[...] + y1a_ref[...]) * dinv
        aggB = (aggB_ref[...] + y1b_ref[...]) * dinv
        Wz = Wz_ref[...]
        Wr = Wr_ref[...]
        dot = functools.partial(jnp.dot, preferred_element_type=jnp.float32)
        z = jax.nn.sigmoid(dot(aggA, Wz[:D]) + dot(aggB, Wz[D:]) + bz_ref[...])
        r = jax.nn.sigmoid(dot(aggA, Wr[:D]) + dot(aggB, Wr[D:]) + br_ref[...])
        z_ref[...] = z
        y2_ref[...] = dinv * (r * h_ref[...])
        t1_ref[...] = dot(aggA, Wc_ref[:D])

    tc2 = pl.pallas_call(
        tc2_body,
        grid=(TCGRID,),
        in_specs=[_row_spec(D)] * 6
        + [_full_spec(2 * D, D)] * 3
        + [_full_spec(1, D)] * 2,
        out_specs=[_row_spec(D)] * 3,
        out_shape=[_f32((NPAD, D))] * 3,
    )

    def tc3_body(p0_ref, p1_ref, y2_ref, dinv_ref, t1_ref, z_ref, h_ref,
                 Wc_ref, bc_ref, lnw_ref, lnb_ref, out_ref):
        dot = functools.partial(jnp.dot, preferred_element_type=jnp.float32)
        agg2 = (p0_ref[...] + p1_ref[...] + y2_ref[...]) * dinv_ref[...]
        cand = jnp.tanh(t1_ref[...] + dot(agg2, Wc_ref[D:]) + bc_ref[...])
        z = z_ref[...]
        hn = (1.0 - z) * h_ref[...] + z * cand
        mu = jnp.mean(hn, axis=1, keepdims=True)
        var = jnp.mean((hn - mu) ** 2, axis=1, keepdims=True)
        out_ref[...] = (hn - mu) * lax.rsqrt(var + EPS) * lnw_ref[...] + lnb_ref[...]

    tc3 = pl.pallas_call(
        tc3_body,
        grid=(TCGRID,),
        in_specs=[_row_spec(D)] * 7
        + [_full_spec(2 * D, D)]
        + [_full_spec(1, D)] * 3,
        out_specs=_row_spec(D),
        out_shape=_f32((NPAD, D)),
    )

    return sc_degree, edge_pass_b, edge_pass_c, tc1, tc2, tc3


def kernel(x, edge_index, h_prev, Wz, bz, Wr, br, Wc, bc, ln_w, ln_b):
    sc_degree, edge_pass_b, edge_pass_c, tc1, tc2, tc3 = _build()
    bsz, n, _ = x.shape
    x2 = x.reshape(n, D)
    h2 = h_prev.reshape(n, D)
    xp = jnp.pad(x2, ((0, NPAD - n), (0, 0)))
    hp = jnp.pad(h2, ((0, NPAD - n), (0, 0)))
    src = jnp.pad(edge_index[0], (0, EPAD - E), constant_values=n)
    dst = jnp.pad(edge_index[1], (0, EPAD - E), constant_values=n)
    src = src.reshape(EROWS, ECHUNK)
    dst = dst.reshape(EROWS, ECHUNK)

    degp = sc_degree(dst)
    y1a, y1b, dinvc = tc1(degp.reshape(NSC * NTILE, NPAD).T, xp, hp)
    y1s = jnp.concatenate([y1a, y1b], axis=0)
    aggB = edge_pass_b(y1s, src, dst)
    z, y2, t1 = tc2(aggB[0], aggB[1], y1a, y1b, dinvc, hp, Wz, Wr, Wc,
                    bz.reshape(1, D), br.reshape(1, D))
    aggC = edge_pass_c(y2, src, dst)
    hout = tc3(aggC[0], aggC[1], y2, dinvc, t1, z, hp, Wc,
               bc.reshape(1, D), ln_w.reshape(1, D), ln_b.reshape(1, D))
    return hout[:n].reshape(bsz, n, D)
